# baseline (speedup 1.0000x reference)
import functools

import jax
import jax.numpy as jnp
from jax.experimental import pallas as pl
from jax.experimental.pallas import tpu as pltpu

EPS = 1e-5


def _round_up(v, m):
    return (v + m - 1) // m * m


def _bn(y, gamma, beta, n_rows, *, relu):
    inv_n = 1.0 / n_rows
    mean = jnp.sum(y, axis=0, keepdims=True) * inv_n
    var = jnp.sum(y * y, axis=0, keepdims=True) * inv_n - mean * mean
    var = jnp.maximum(var, 0.0)
    scale = jax.lax.rsqrt(var + EPS) * gamma
    out = (y - mean) * scale + beta
    return jnp.maximum(out, 0.0) if relu else out


def _fused_kernel(x_ref, w1_ref, w2_ref, w3_ref,
                  g1_ref, b1_ref, g2_ref, b2_ref,
                  o_ref, xp_ref, pt_ref, *, N, H, W, pad_off):
    HW = H * W
    M = N * HW
    Mpad, Cmid = xp_ref.shape

    y1 = jnp.dot(x_ref[...], w1_ref[...],
                 preferred_element_type=jnp.float32)
    z1 = _bn(y1, g1_ref[...], b1_ref[...], M, relu=True)
    xp_ref[0:pad_off, :] = jnp.zeros((pad_off, Cmid), xp_ref.dtype)
    xp_ref[pad_off + M:Mpad, :] = jnp.zeros((Mpad - pad_off - M, Cmid),
                                            xp_ref.dtype)
    xp_ref[pad_off:pad_off + M, :] = z1.astype(jnp.bfloat16)

    ii = jax.lax.broadcasted_iota(jnp.int32, (M, 1), 0)
    yy = (ii % HW) // W
    xx = ii % W
    row_ok = {-1: yy >= 1, 0: None, 1: yy < H - 1}
    col_ok = {-1: xx >= 1, 0: None, 1: xx < W - 1}

    for dy in (-1, 0, 1):
        for dx in (-1, 0, 1):
            t = (dy + 1) * 3 + (dx + 1)
            start = pad_off + dy * W + dx
            patch = xp_ref[start:start + M, :]
            if dy == 0 and dx == 0:
                ok = None
            elif dy == 0:
                ok = col_ok[dx]
            elif dx == 0:
                ok = row_ok[dy]
            else:
                ok = jnp.logical_and(row_ok[dy], col_ok[dx])
            if ok is not None:
                patch = jnp.where(ok, patch, jnp.bfloat16(0))
            pt_ref[:, t * Cmid:(t + 1) * Cmid] = patch

    acc = jnp.dot(pt_ref[...], w2_ref[...],
                  preferred_element_type=jnp.float32)

    z2 = _bn(acc, g1_ref[...], b1_ref[...], M, relu=True)
    y3 = jnp.dot(z2.astype(jnp.bfloat16), w3_ref[...],
                 preferred_element_type=jnp.float32)

    inv_n = 1.0 / M
    mean = jnp.sum(y3, axis=0, keepdims=True) * inv_n
    var = jnp.sum(y3 * y3, axis=0, keepdims=True) * inv_n - mean * mean
    var = jnp.maximum(var, 0.0)
    scale = jax.lax.rsqrt(var + EPS) * g2_ref[...]
    shift = b2_ref[...] - mean * scale
    o_ref[...] = jnp.maximum(y3 * scale + shift
                             + x_ref[...].astype(jnp.float32), 0.0)


@jax.jit
def _forward(x_nchw, w1, w2, w3, g1, b1, g2, b2):
    N, Cin, H, W = x_nchw.shape
    Cin_p, Cmid_p = w1.shape
    assert Cin == Cin_p, "lane-padding for Cin not needed at these shapes"
    HW = H * W
    M = N * HW
    pad_off = _round_up(W + 1, 8)
    Mpad = _round_up(pad_off + M + W + 1, 8)

    x_flat = jnp.transpose(x_nchw, (0, 2, 3, 1)).reshape(M, Cin).astype(
        jnp.bfloat16)
    w2_flat = w2.reshape(9 * Cmid_p, Cmid_p)

    out = pl.pallas_call(
        functools.partial(_fused_kernel, N=N, H=H, W=W, pad_off=pad_off),
        out_shape=jax.ShapeDtypeStruct((M, Cin), jnp.float32),
        grid=(1,),
        in_specs=[
            pl.BlockSpec((M, Cin_p), lambda g: (0, 0)),
            pl.BlockSpec((Cin_p, Cmid_p), lambda g: (0, 0)),
            pl.BlockSpec((9 * Cmid_p, Cmid_p), lambda g: (0, 0)),
            pl.BlockSpec((Cmid_p, Cin_p), lambda g: (0, 0)),
            pl.BlockSpec((1, Cmid_p), lambda g: (0, 0)),
            pl.BlockSpec((1, Cmid_p), lambda g: (0, 0)),
            pl.BlockSpec((1, Cin_p), lambda g: (0, 0)),
            pl.BlockSpec((1, Cin_p), lambda g: (0, 0)),
        ],
        out_specs=pl.BlockSpec((M, Cin_p), lambda g: (0, 0)),
        scratch_shapes=[
            pltpu.VMEM((Mpad, Cmid_p), jnp.bfloat16),
            pltpu.VMEM((M, 9 * Cmid_p), jnp.bfloat16),
        ],
        compiler_params=pltpu.CompilerParams(
            dimension_semantics=("arbitrary",),
            vmem_limit_bytes=56 << 20,
        ),
    )(x_flat, w1, w2_flat, w3, g1, b1, g2, b2)

    y = out.reshape(N, H, W, Cin)
    return jnp.transpose(y, (0, 3, 1, 2))


def kernel(x, w1, w2, w3, g1, b1, g2, b2):
    return _forward(x, w1, w2, w3, g1, b1, g2, b2)


# speedup vs baseline: 1.0321x; 1.0321x over previous
import functools

import jax
import jax.numpy as jnp
from jax.experimental import pallas as pl
from jax.experimental.pallas import tpu as pltpu

EPS = 1e-5


def _round_up(v, m):
    return (v + m - 1) // m * m


def _bn(y, gamma, beta, n_rows, *, relu):
    inv_n = 1.0 / n_rows
    mean = jnp.sum(y, axis=0, keepdims=True) * inv_n
    var = jnp.sum(y * y, axis=0, keepdims=True) * inv_n - mean * mean
    var = jnp.maximum(var, 0.0)
    scale = jax.lax.rsqrt(var + EPS) * gamma
    out = (y - mean) * scale + beta
    return jnp.maximum(out, 0.0) if relu else out


def _fused_kernel(x_hbm, w1_ref, w2_hbm, w3_hbm,
                  g1_ref, b1_ref, g2_ref, b2_ref,
                  o_ref,
                  xv_ref, w2v_ref, w3v_ref, y1_ref, xp_ref,
                  sx0, sx1, sx2, sx3, sw2, sw3,
                  *, N, H, W, pad_off, chunks):
    HW = H * W
    M = N * HW
    Mpad, Cmid = xp_ref.shape

    xsems = (sx0, sx1, sx2, sx3)
    xdmas = []
    for k, (s, sz) in enumerate(chunks):
        dma = pltpu.make_async_copy(x_hbm.at[pl.ds(s, sz), :],
                                    xv_ref.at[pl.ds(s, sz), :], xsems[k])
        dma.start()
        xdmas.append(dma)
    w2dma = pltpu.make_async_copy(w2_hbm, w2v_ref, sw2)
    w2dma.start()
    w3dma = pltpu.make_async_copy(w3_hbm, w3v_ref, sw3)
    w3dma.start()

    for k, (s, sz) in enumerate(chunks):
        xdmas[k].wait()
        y1_ref[s:s + sz, :] = jnp.dot(xv_ref[s:s + sz, :], w1_ref[...],
                                      preferred_element_type=jnp.float32)

    z1 = _bn(y1_ref[0:M, :], g1_ref[...], b1_ref[...], M, relu=True)
    xp_ref[0:pad_off, :] = jnp.zeros((pad_off, Cmid), xp_ref.dtype)
    xp_ref[pad_off + M:Mpad, :] = jnp.zeros((Mpad - pad_off - M, Cmid),
                                            xp_ref.dtype)
    xp_ref[pad_off:pad_off + M, :] = z1.astype(jnp.bfloat16)

    ii = jax.lax.broadcasted_iota(jnp.int32, (M, 1), 0)
    yy = (ii % HW) // W
    xx = ii % W
    row_ok = {-1: yy >= 1, 1: yy < H - 1}
    col_ok = {-1: xx >= 1, 1: xx < W - 1}

    w2dma.wait()
    acc = jnp.dot(xp_ref[pad_off:pad_off + M, :], w2v_ref[1, 1, :, :],
                  preferred_element_type=jnp.float32)
    for dy in (-1, 0, 1):
        for dx in (-1, 0, 1):
            if dy == 0 and dx == 0:
                continue
            start = pad_off + dy * W + dx
            if dy == 0:
                ok = col_ok[dx]
            elif dx == 0:
                ok = row_ok[dy]
            else:
                ok = jnp.logical_and(row_ok[dy], col_ok[dx])
            patch = jnp.where(ok, xp_ref[start:start + M, :], jnp.bfloat16(0))
            acc = acc + jnp.dot(patch, w2v_ref[dy + 1, dx + 1, :, :],
                                preferred_element_type=jnp.float32)

    z2 = _bn(acc, g1_ref[...], b1_ref[...], M, relu=True)
    w3dma.wait()
    y3 = jnp.dot(z2.astype(jnp.bfloat16), w3v_ref[...],
                 preferred_element_type=jnp.float32)

    inv_n = 1.0 / M
    mean = jnp.sum(y3, axis=0, keepdims=True) * inv_n
    var = jnp.sum(y3 * y3, axis=0, keepdims=True) * inv_n - mean * mean
    var = jnp.maximum(var, 0.0)
    scale = jax.lax.rsqrt(var + EPS) * g2_ref[...]
    shift = b2_ref[...] - mean * scale
    o_ref[...] = jnp.maximum(y3 * scale + shift
                             + xv_ref[...].astype(jnp.float32), 0.0)


@jax.jit
def _forward(x_nchw, w1, w2, w3, g1, b1, g2, b2):
    N, Cin, H, W = x_nchw.shape
    Cin_p, Cmid_p = w1.shape
    assert Cin == Cin_p, "lane-padding for Cin not needed at these shapes"
    HW = H * W
    M = N * HW
    pad_off = _round_up(W + 1, 8)
    Mpad = _round_up(pad_off + M + W + 1, 8)

    n_chunks = 4
    step = _round_up(-(-M // n_chunks), 8)
    chunks = tuple((s, min(step, M - s)) for s in range(0, M, step))

    x_flat = jnp.transpose(x_nchw, (0, 2, 3, 1)).reshape(M, Cin).astype(
        jnp.bfloat16)

    out = pl.pallas_call(
        functools.partial(_fused_kernel, N=N, H=H, W=W, pad_off=pad_off,
                          chunks=chunks),
        out_shape=jax.ShapeDtypeStruct((M, Cin), jnp.float32),
        grid=(1,),
        in_specs=[
            pl.BlockSpec(memory_space=pl.ANY),
            pl.BlockSpec((Cin_p, Cmid_p), lambda g: (0, 0)),
            pl.BlockSpec(memory_space=pl.ANY),
            pl.BlockSpec(memory_space=pl.ANY),
            pl.BlockSpec((1, Cmid_p), lambda g: (0, 0)),
            pl.BlockSpec((1, Cmid_p), lambda g: (0, 0)),
            pl.BlockSpec((1, Cin_p), lambda g: (0, 0)),
            pl.BlockSpec((1, Cin_p), lambda g: (0, 0)),
        ],
        out_specs=pl.BlockSpec((M, Cin_p), lambda g: (0, 0)),
        scratch_shapes=[
            pltpu.VMEM((M, Cin_p), jnp.bfloat16),
            pltpu.VMEM((3, 3, Cmid_p, Cmid_p), jnp.bfloat16),
            pltpu.VMEM((Cmid_p, Cin_p), jnp.bfloat16),
            pltpu.VMEM((M, Cmid_p), jnp.float32),
            pltpu.VMEM((Mpad, Cmid_p), jnp.bfloat16),
            pltpu.SemaphoreType.DMA,
            pltpu.SemaphoreType.DMA,
            pltpu.SemaphoreType.DMA,
            pltpu.SemaphoreType.DMA,
            pltpu.SemaphoreType.DMA,
            pltpu.SemaphoreType.DMA,
        ],
        compiler_params=pltpu.CompilerParams(
            dimension_semantics=("arbitrary",),
            vmem_limit_bytes=56 << 20,
        ),
    )(x_flat, w1, w2, w3, g1, b1, g2, b2)

    y = out.reshape(N, H, W, Cin)
    return jnp.transpose(y, (0, 3, 1, 2))


def kernel(x, w1, w2, w3, g1, b1, g2, b2):
    return _forward(x, w1, w2, w3, g1, b1, g2, b2)


# speedup vs baseline: 1.0878x; 1.0540x over previous
import functools

import jax
import jax.numpy as jnp
from jax.experimental import pallas as pl
from jax.experimental.pallas import tpu as pltpu

EPS = 1e-5


def _round_up(v, m):
    return (v + m - 1) // m * m


def _bn(y, gamma, beta, n_rows, *, relu):
    inv_n = 1.0 / n_rows
    mean = jnp.sum(y, axis=0, keepdims=True) * inv_n
    var = jnp.sum(y * y, axis=0, keepdims=True) * inv_n - mean * mean
    var = jnp.maximum(var, 0.0)
    scale = jax.lax.rsqrt(var + EPS) * gamma
    out = (y - mean) * scale + beta
    return jnp.maximum(out, 0.0) if relu else out


def _fused_kernel(x_hbm, w1_ref, w2_hbm, w3_hbm,
                  g1_ref, b1_ref, g2_ref, b2_ref,
                  o_ref,
                  xv_ref, w2v_ref, w3v_ref, y1_ref, xp_ref,
                  sx0, sx1, sx2, sx3, sw2, sw3,
                  *, N, H, W, pad_off, chunks):
    HW = H * W
    M = N * HW
    Mpad, Cmid = xp_ref.shape

    xsems = (sx0, sx1, sx2, sx3)
    xdmas = []
    for k, (s, sz) in enumerate(chunks):
        dma = pltpu.make_async_copy(x_hbm.at[pl.ds(s, sz), :],
                                    xv_ref.at[pl.ds(s, sz), :], xsems[k])
        dma.start()
        xdmas.append(dma)
    w2dma = pltpu.make_async_copy(w2_hbm, w2v_ref, sw2)
    w2dma.start()
    w3dma = pltpu.make_async_copy(w3_hbm, w3v_ref, sw3)
    w3dma.start()

    for k, (s, sz) in enumerate(chunks):
        xdmas[k].wait()
        y1_ref[s:s + sz, :] = jnp.dot(xv_ref[s:s + sz, :], w1_ref[...],
                                      preferred_element_type=jnp.float32)

    z1 = _bn(y1_ref[0:M, :], g1_ref[...], b1_ref[...], M, relu=True)
    xp_ref[0:pad_off, :] = jnp.zeros((pad_off, Cmid), xp_ref.dtype)
    xp_ref[pad_off + M:Mpad, :] = jnp.zeros((Mpad - pad_off - M, Cmid),
                                            xp_ref.dtype)
    xp_ref[pad_off:pad_off + M, :] = z1.astype(jnp.bfloat16)

    ii = jax.lax.broadcasted_iota(jnp.int32, (M, 1), 0)
    yy = (ii % HW) // W
    xx = ii % W
    row_ok = {-1: yy >= 1, 1: yy < H - 1}
    col_ok = {-1: xx >= 1, 1: xx < W - 1}

    w2dma.wait()
    acc = jnp.dot(xp_ref[pad_off:pad_off + M, :], w2v_ref[1, 1, :, :],
                  preferred_element_type=jnp.float32)
    for dy in (-1, 0, 1):
        for dx in (-1, 0, 1):
            if dy == 0 and dx == 0:
                continue
            start = pad_off + dy * W + dx
            if dy == 0:
                ok = col_ok[dx]
            elif dx == 0:
                ok = row_ok[dy]
            else:
                ok = jnp.logical_and(row_ok[dy], col_ok[dx])
            patch = jnp.where(ok, xp_ref[start:start + M, :], jnp.bfloat16(0))
            acc = acc + jnp.dot(patch, w2v_ref[dy + 1, dx + 1, :, :],
                                preferred_element_type=jnp.float32)

    z2 = _bn(acc, g1_ref[...], b1_ref[...], M, relu=True)
    w3dma.wait()
    y3 = jnp.dot(z2.astype(jnp.bfloat16), w3v_ref[...],
                 preferred_element_type=jnp.float32)

    inv_n = 1.0 / M
    mean = jnp.sum(y3, axis=0, keepdims=True) * inv_n
    var = jnp.sum(y3 * y3, axis=0, keepdims=True) * inv_n - mean * mean
    var = jnp.maximum(var, 0.0)
    scale = jax.lax.rsqrt(var + EPS) * g2_ref[...]
    shift = b2_ref[...] - mean * scale
    o_ref[...] = jnp.maximum(y3 * scale + shift
                             + xv_ref[...].astype(jnp.float32),
                             0.0).astype(jnp.bfloat16)


@jax.jit
def _forward(x_nchw, w1, w2, w3, g1, b1, g2, b2):
    N, Cin, H, W = x_nchw.shape
    Cin_p, Cmid_p = w1.shape
    assert Cin == Cin_p, "lane-padding for Cin not needed at these shapes"
    HW = H * W
    M = N * HW
    pad_off = _round_up(W + 1, 8)
    Mpad = _round_up(pad_off + M + W + 1, 8)

    n_chunks = 4
    step = _round_up(-(-M // n_chunks), 8)
    chunks = tuple((s, min(step, M - s)) for s in range(0, M, step))

    x_flat = jnp.transpose(x_nchw, (0, 2, 3, 1)).reshape(M, Cin).astype(
        jnp.bfloat16)

    out = pl.pallas_call(
        functools.partial(_fused_kernel, N=N, H=H, W=W, pad_off=pad_off,
                          chunks=chunks),
        out_shape=jax.ShapeDtypeStruct((M, Cin), jnp.bfloat16),
        grid=(1,),
        in_specs=[
            pl.BlockSpec(memory_space=pl.ANY),
            pl.BlockSpec((Cin_p, Cmid_p), lambda g: (0, 0)),
            pl.BlockSpec(memory_space=pl.ANY),
            pl.BlockSpec(memory_space=pl.ANY),
            pl.BlockSpec((1, Cmid_p), lambda g: (0, 0)),
            pl.BlockSpec((1, Cmid_p), lambda g: (0, 0)),
            pl.BlockSpec((1, Cin_p), lambda g: (0, 0)),
            pl.BlockSpec((1, Cin_p), lambda g: (0, 0)),
        ],
        out_specs=pl.BlockSpec((M, Cin_p), lambda g: (0, 0)),
        scratch_shapes=[
            pltpu.VMEM((M, Cin_p), jnp.bfloat16),
            pltpu.VMEM((3, 3, Cmid_p, Cmid_p), jnp.bfloat16),
            pltpu.VMEM((Cmid_p, Cin_p), jnp.bfloat16),
            pltpu.VMEM((M, Cmid_p), jnp.float32),
            pltpu.VMEM((Mpad, Cmid_p), jnp.bfloat16),
            pltpu.SemaphoreType.DMA,
            pltpu.SemaphoreType.DMA,
            pltpu.SemaphoreType.DMA,
            pltpu.SemaphoreType.DMA,
            pltpu.SemaphoreType.DMA,
            pltpu.SemaphoreType.DMA,
        ],
        compiler_params=pltpu.CompilerParams(
            dimension_semantics=("arbitrary",),
            vmem_limit_bytes=56 << 20,
        ),
    )(x_flat, w1, w2, w3, g1, b1, g2, b2)

    y = out.reshape(N, H, W, Cin)
    return jnp.transpose(y, (0, 3, 1, 2)).astype(jnp.float32)


def kernel(x, w1, w2, w3, g1, b1, g2, b2):
    return _forward(x, w1, w2, w3, g1, b1, g2, b2)


# speedup vs baseline: 1.0995x; 1.0107x over previous
import functools

import jax
import jax.numpy as jnp
from jax.experimental import pallas as pl
from jax.experimental.pallas import tpu as pltpu

EPS = 1e-5


def _round_up(v, m):
    return (v + m - 1) // m * m


def _bn(y, gamma, beta, n_rows, *, relu):
    inv_n = 1.0 / n_rows
    mean = jnp.sum(y, axis=0, keepdims=True) * inv_n
    var = jnp.sum(y * y, axis=0, keepdims=True) * inv_n - mean * mean
    var = jnp.maximum(var, 0.0)
    scale = jax.lax.rsqrt(var + EPS) * gamma
    out = (y - mean) * scale + beta
    return jnp.maximum(out, 0.0) if relu else out


def _fused_kernel(x_hbm, w1_ref, w2_hbm, w3_hbm,
                  g1_ref, b1_ref, g2_ref, b2_ref,
                  o_ref,
                  xv_ref, w2v_ref, w3v_ref, y1_ref, xp_ref,
                  sx0, sx1, sx2, sx3, sw2, sw3,
                  *, N, H, W, pad_off, chunks):
    HW = H * W
    M = N * HW
    Mpad, Cmid = xp_ref.shape

    xsems = (sx0, sx1, sx2, sx3)
    xdmas = []
    for k, (s, sz) in enumerate(chunks):
        dma = pltpu.make_async_copy(x_hbm.at[pl.ds(s, sz), :],
                                    xv_ref.at[pl.ds(s, sz), :], xsems[k])
        dma.start()
        xdmas.append(dma)
    w2dma = pltpu.make_async_copy(w2_hbm, w2v_ref, sw2)
    w2dma.start()
    w3dma = pltpu.make_async_copy(w3_hbm, w3v_ref, sw3)
    w3dma.start()

    for k, (s, sz) in enumerate(chunks):
        xdmas[k].wait()
        y1_ref[s:s + sz, :] = jnp.dot(xv_ref[s:s + sz, :], w1_ref[...],
                                      preferred_element_type=jnp.float32)

    z1 = _bn(y1_ref[0:M, :], g1_ref[...], b1_ref[...], M, relu=True)
    xp_ref[0:pad_off, :] = jnp.zeros((pad_off, Cmid), xp_ref.dtype)
    xp_ref[pad_off + M:Mpad, :] = jnp.zeros((Mpad - pad_off - M, Cmid),
                                            xp_ref.dtype)
    xp_ref[pad_off:pad_off + M, :] = z1.astype(jnp.bfloat16)

    ii = jax.lax.broadcasted_iota(jnp.int32, (M, 1), 0)
    yy = (ii % HW) // W
    xx = ii % W
    row_ok = {-1: yy >= 1, 1: yy < H - 1}
    col_ok = {-1: xx >= 1, 1: xx < W - 1}

    w2dma.wait()
    acc = jnp.dot(xp_ref[pad_off:pad_off + M, :], w2v_ref[1, 1, :, :],
                  preferred_element_type=jnp.float32)
    for dy in (-1, 0, 1):
        for dx in (-1, 0, 1):
            if dy == 0 and dx == 0:
                continue
            start = pad_off + dy * W + dx
            if dy == 0:
                ok = col_ok[dx]
            elif dx == 0:
                ok = row_ok[dy]
            else:
                ok = jnp.logical_and(row_ok[dy], col_ok[dx])
            patch = jnp.where(ok, xp_ref[start:start + M, :], jnp.bfloat16(0))
            acc = acc + jnp.dot(patch, w2v_ref[dy + 1, dx + 1, :, :],
                                preferred_element_type=jnp.float32)

    z2 = _bn(acc, g1_ref[...], b1_ref[...], M, relu=True)
    w3dma.wait()
    y3 = jnp.dot(z2.astype(jnp.bfloat16), w3v_ref[...],
                 preferred_element_type=jnp.float32)

    inv_n = 1.0 / M
    mean = jnp.sum(y3, axis=0, keepdims=True) * inv_n
    var = jnp.sum(y3 * y3, axis=0, keepdims=True) * inv_n - mean * mean
    var = jnp.maximum(var, 0.0)
    scale = jax.lax.rsqrt(var + EPS) * g2_ref[...]
    shift = b2_ref[...] - mean * scale
    yb = (y3 * scale + shift).astype(jnp.bfloat16)
    o_ref[...] = jnp.maximum(yb + xv_ref[...], jnp.bfloat16(0))


@jax.jit
def _forward(x_nchw, w1, w2, w3, g1, b1, g2, b2):
    N, Cin, H, W = x_nchw.shape
    Cin_p, Cmid_p = w1.shape
    assert Cin == Cin_p, "lane-padding for Cin not needed at these shapes"
    HW = H * W
    M = N * HW
    pad_off = _round_up(W + 1, 8)
    Mpad = _round_up(pad_off + M + W + 1, 8)

    n_chunks = 4
    step = _round_up(-(-M // n_chunks), 8)
    chunks = tuple((s, min(step, M - s)) for s in range(0, M, step))

    x_flat = jnp.transpose(x_nchw, (0, 2, 3, 1)).reshape(M, Cin).astype(
        jnp.bfloat16)

    out = pl.pallas_call(
        functools.partial(_fused_kernel, N=N, H=H, W=W, pad_off=pad_off,
                          chunks=chunks),
        out_shape=jax.ShapeDtypeStruct((M, Cin), jnp.bfloat16),
        grid=(1,),
        in_specs=[
            pl.BlockSpec(memory_space=pl.ANY),
            pl.BlockSpec((Cin_p, Cmid_p), lambda g: (0, 0)),
            pl.BlockSpec(memory_space=pl.ANY),
            pl.BlockSpec(memory_space=pl.ANY),
            pl.BlockSpec((1, Cmid_p), lambda g: (0, 0)),
            pl.BlockSpec((1, Cmid_p), lambda g: (0, 0)),
            pl.BlockSpec((1, Cin_p), lambda g: (0, 0)),
            pl.BlockSpec((1, Cin_p), lambda g: (0, 0)),
        ],
        out_specs=pl.BlockSpec((M, Cin_p), lambda g: (0, 0)),
        scratch_shapes=[
            pltpu.VMEM((M, Cin_p), jnp.bfloat16),
            pltpu.VMEM((3, 3, Cmid_p, Cmid_p), jnp.bfloat16),
            pltpu.VMEM((Cmid_p, Cin_p), jnp.bfloat16),
            pltpu.VMEM((M, Cmid_p), jnp.float32),
            pltpu.VMEM((Mpad, Cmid_p), jnp.bfloat16),
            pltpu.SemaphoreType.DMA,
            pltpu.SemaphoreType.DMA,
            pltpu.SemaphoreType.DMA,
            pltpu.SemaphoreType.DMA,
            pltpu.SemaphoreType.DMA,
            pltpu.SemaphoreType.DMA,
        ],
        compiler_params=pltpu.CompilerParams(
            dimension_semantics=("arbitrary",),
            vmem_limit_bytes=56 << 20,
        ),
    )(x_flat, w1, w2, w3, g1, b1, g2, b2)

    y = out.reshape(N, H, W, Cin)
    return jnp.transpose(y, (0, 3, 1, 2)).astype(jnp.float32)


def kernel(x, w1, w2, w3, g1, b1, g2, b2):
    return _forward(x, w1, w2, w3, g1, b1, g2, b2)
